# Initial kernel scaffold; baseline (speedup 1.0000x reference)
import jax
import jax.numpy as jnp
from jax.experimental import pallas as pl


def kernel(features, indices):
    raise NotImplementedError("write your pallas kernel here")


# speedup vs baseline: 1.4345x; 1.4345x over previous
import functools

import jax
import jax.numpy as jnp
from jax import lax
from jax.experimental import pallas as pl
from jax.experimental.pallas import tpu as pltpu
from jax.experimental.pallas import tpu_sc as plsc

_N, _C, _D, _H, _W = 4, 128, 2, 200, 176
_NNZ = 150000
_DH = _D * _H
_NROWS = _N * _DH
_NDHW = _NROWS * _W
_NC = 2
_NS = 16
_NW = _NC * _NS
_SLOTS_W = _NDHW // _NW
_ROWS_W = _SLOTS_W // _W
_CHUNK = 6000
_NCHUNK = _NNZ // _CHUNK
_QPC = _CHUNK // 16
_GSUB = 88


def _hc_body(feat_hbm, idx_hbm, out_hbm, map_v, idxb, gidx, rows_v, tb, sem):
    wid = lax.axis_index("s") * _NC + lax.axis_index("c")
    lo = wid * _SLOTS_W
    iota16 = lax.broadcasted_iota(jnp.int32, (16,), 0)

    def init_body(i, _):
        map_v[pl.ds(i * 16, 16)] = jnp.full((16,), -1, jnp.int32)
        return 0

    lax.fori_loop(0, _SLOTS_W // 16, init_body, 0)

    def chunk_body(c, _):
        pltpu.sync_copy(idx_hbm.at[pl.ds(c * _CHUNK, _CHUNK)], idxb)

        def q_body(q, _):
            g = idxb[pl.ds(q * 16, 16)]
            m = (g >= lo) & (g < lo + _SLOTS_W)
            local = jnp.where(m, g - lo, 0)
            vid = (c * _CHUNK + q * 16) + iota16
            plsc.store_scatter(map_v, [local], vid, mask=m)
            return 0

        lax.fori_loop(0, _QPC, q_body, 0)
        return 0

    lax.fori_loop(0, _NCHUNK, chunk_body, 0)

    def row_body(r, _):
        grow = wid * _ROWS_W + r
        n = grow // _DH
        dh = grow - n * _DH
        base = r * _W

        def gi_body(q, _):
            off = base + q * 16
            mv = map_v[pl.ds(off, 16)]
            sent = (off + iota16) & 63
            gidx[pl.ds(q * 16, 16)] = jnp.where(mv < 0, sent, mv)
            return 0

        lax.fori_loop(0, _W // 16, gi_body, 0)

        cp1 = pltpu.make_async_copy(
            feat_hbm.at[gidx.at[pl.ds(0, _GSUB)]], rows_v.at[pl.ds(0, _GSUB)], sem)
        cp2 = pltpu.make_async_copy(
            feat_hbm.at[gidx.at[pl.ds(_GSUB, _GSUB)]],
            rows_v.at[pl.ds(_GSUB, _GSUB)], sem)
        cp1.start()
        cp2.start()
        cp1.wait()
        cp2.wait()

        def tq_body(q, _):
            mvec = map_v[pl.ds(base + q * 16, 16)]
            full = mvec >= 0
            kvec = q * 16 + iota16
            for c in range(_C):
                cvec = jnp.full((16,), c, jnp.int32)
                v = plsc.load_gather(rows_v, [kvec, cvec])
                v = jnp.where(full, v, 0.0)
                tb[c, pl.ds(q * 16, 16)] = v
            return 0

        lax.fori_loop(0, _W // 16, tq_body, 0)

        pltpu.sync_copy(tb, out_hbm.at[n, :, dh, :])
        return 0

    lax.fori_loop(0, _ROWS_W, row_body, 0)


_hc_kernel = functools.partial(
    pl.kernel,
    out_type=jax.ShapeDtypeStruct((_N, _C, _DH, _W), jnp.float32),
    mesh=plsc.VectorSubcoreMesh(core_axis_name="c", subcore_axis_name="s"),
    scratch_types=[
        pltpu.VMEM((_SLOTS_W,), jnp.int32),
        pltpu.VMEM((_CHUNK,), jnp.int32),
        pltpu.VMEM((_W,), jnp.int32),
        pltpu.VMEM((_W, _C), jnp.float32),
        pltpu.VMEM((_C, _W), jnp.float32),
        pltpu.SemaphoreType.DMA,
    ],
    compiler_params=pltpu.CompilerParams(needs_layout_passes=False),
)(_hc_body)


def kernel(features, indices):
    out = _hc_kernel(features, indices)
    return out.reshape(_N, _C * _D, _H, _W)


# speedup vs baseline: 2.1185x; 1.4769x over previous
import functools

import jax
import jax.numpy as jnp
from jax import lax
from jax.experimental import pallas as pl
from jax.experimental.pallas import tpu as pltpu
from jax.experimental.pallas import tpu_sc as plsc

_N, _C, _D, _H, _W = 4, 128, 2, 200, 176
_NNZ = 150000
_DH = _D * _H
_NROWS = _N * _DH
_NDHW = _NROWS * _W
_NC = 2
_NS = 16
_NW = _NC * _NS
_SLOTS_W = _NDHW // _NW
_ROWS_W = _SLOTS_W // _W
_CHUNK = 6000
_NCHUNK = _NNZ // _CHUNK
_QPC = _CHUNK // 16
_GSUB = 88


def _hc_body(feat_hbm, idx_hbm, out_hbm, map_v, idxb, gidx, rows_v, tb, sem):
    wid = lax.axis_index("s") * _NC + lax.axis_index("c")
    lo = wid * _SLOTS_W
    iota16 = lax.broadcasted_iota(jnp.int32, (16,), 0)

    def init_body(i, _):
        map_v[pl.ds(i * 16, 16)] = jnp.full((16,), -1, jnp.int32)
        return 0

    lax.fori_loop(0, _SLOTS_W // 16, init_body, 0)

    def chunk_body(c, _):
        pltpu.sync_copy(idx_hbm.at[pl.ds(c * _CHUNK, _CHUNK)], idxb)

        def q_body(q, _):
            g = idxb[pl.ds(q * 16, 16)]
            m = (g >= lo) & (g < lo + _SLOTS_W)
            local = jnp.where(m, g - lo, 0)
            vid = (c * _CHUNK + q * 16) + iota16
            plsc.store_scatter(map_v, [local], vid, mask=m)
            return 0

        lax.fori_loop(0, _QPC, q_body, 0, unroll=4)
        return 0

    lax.fori_loop(0, _NCHUNK, chunk_body, 0)

    ci = [c0 + iota16 for c0 in range(0, _C, 16)]
    zero16 = jnp.zeros((16,), jnp.float32)

    def row_body(r, _):
        grow = wid * _ROWS_W + r
        n = grow // _DH
        dh = grow - n * _DH
        base = r * _W

        cp1 = pltpu.make_async_copy(
            feat_hbm.at[plsc.Indices(map_v.at[pl.ds(base, _GSUB)],
                                     ignored_value=-1)],
            rows_v.at[pl.ds(0, _GSUB)], sem)
        cp2 = pltpu.make_async_copy(
            feat_hbm.at[plsc.Indices(map_v.at[pl.ds(base + _GSUB, _GSUB)],
                                     ignored_value=-1)],
            rows_v.at[pl.ds(_GSUB, _GSUB)], sem)
        cp1.start()
        cp2.start()
        cp1.wait()
        cp2.wait()

        def tq_body(q, _):
            mvec = map_v[pl.ds(base + q * 16, 16)]
            for j in range(16):
                k = q * 16 + j
                kvec = jnp.full((16,), 0, jnp.int32) + k

                @pl.when(mvec[j] >= 0)
                def _():
                    for i in range(_C // 16):
                        v = rows_v[k, pl.ds(i * 16, 16)]
                        plsc.store_scatter(tb, [ci[i], kvec], v)

                @pl.when(mvec[j] < 0)
                def _():
                    for i in range(_C // 16):
                        plsc.store_scatter(tb, [ci[i], kvec], zero16)

            return 0

        lax.fori_loop(0, _W // 16, tq_body, 0)

        pltpu.sync_copy(tb.at[:, pl.ds(0, _W)], out_hbm.at[n, :, dh, :])
        return 0

    lax.fori_loop(0, _ROWS_W, row_body, 0)


_hc_kernel = functools.partial(
    pl.kernel,
    out_type=jax.ShapeDtypeStruct((_N, _C, _DH, _W), jnp.float32),
    mesh=plsc.VectorSubcoreMesh(core_axis_name="c", subcore_axis_name="s"),
    scratch_types=[
        pltpu.VMEM((_SLOTS_W,), jnp.int32),
        pltpu.VMEM((_CHUNK,), jnp.int32),
        pltpu.VMEM((_W,), jnp.int32),
        pltpu.VMEM((_W, _C), jnp.float32),
        pltpu.VMEM((_C, _W + 1), jnp.float32),
        pltpu.SemaphoreType.DMA,
    ],
    compiler_params=pltpu.CompilerParams(
        needs_layout_passes=False, use_tc_tiling_on_sc=False),
)(_hc_body)


def kernel(features, indices):
    out = _hc_kernel(features, indices)
    return out.reshape(_N, _C * _D, _H, _W)


# speedup vs baseline: 3.7540x; 1.7720x over previous
import functools

import jax
import jax.numpy as jnp
from jax import lax
from jax.experimental import pallas as pl
from jax.experimental.pallas import tpu as pltpu
from jax.experimental.pallas import tpu_sc as plsc

_N, _C, _D, _H, _W = 4, 128, 2, 200, 176
_CD = _C * _D
_HW = _H * _W
_NNZ = 150000
_NC = 2
_NS = 16
_NW = _NC * _NS
_HB = _H // (_NW // _N)
_RW = _HB * _W
_CHUNK = 6000
_NCHUNK = _NNZ // _CHUNK
_QPC = _CHUNK // 16
_GSUB = 88


def _hc_body(feat_hbm, idx_hbm, out_hbm, map_v, idxb, rows_v, ob, sem):
    wid = lax.axis_index("s") * _NC + lax.axis_index("c")
    n = wid // (_NW // _N)
    hb = wid - n * (_NW // _N)
    base0 = n * (_D * _HW) + hb * _RW
    base1 = base0 + _HW
    iota16 = lax.broadcasted_iota(jnp.int32, (16,), 0)

    def init_body(i, _):
        map_v[pl.ds(i * 16, 16)] = jnp.full((16,), -1, jnp.int32)
        return 0

    lax.fori_loop(0, 2 * _RW // 16, init_body, 0)

    def chunk_body(c, _):
        pltpu.sync_copy(idx_hbm.at[pl.ds(c * _CHUNK, _CHUNK)], idxb)

        def q_body(q, _):
            g = idxb[pl.ds(q * 16, 16)]
            in0 = (g >= base0) & (g < base0 + _RW)
            in1 = (g >= base1) & (g < base1 + _RW)
            m = in0 | in1
            local = jnp.where(in0, g - base0, g - (base1 - _RW))
            local = jnp.where(m, local, 0)
            vid = (c * _CHUNK + q * 16) + iota16
            plsc.store_scatter(map_v, [local], vid, mask=m)
            return 0

        lax.fori_loop(0, _QPC, q_body, 0, unroll=4)
        return 0

    lax.fori_loop(0, _NCHUNK, chunk_body, 0)

    ch_idx = [[(i * 16 + iota16) * _D + d for i in range(_C // 16)]
              for d in range(_D)]
    zero16 = jnp.zeros((16,), jnp.float32)

    def row_body(r, _):
        h = hb * _HB + r

        def z_body(w, _):
            for half in range(2):
                for i in range(_C // 16):
                    rows_v[_W * half + w, pl.ds(i * 16, 16)] = zero16
            return 0

        lax.fori_loop(0, _W, z_body, 0)

        cps = []
        for mo, dst in ((r * _W, 0), (r * _W + _GSUB, _GSUB),
                        (_RW + r * _W, _W), (_RW + r * _W + _GSUB, _W + _GSUB)):
            cp = pltpu.make_async_copy(
                feat_hbm.at[plsc.Indices(map_v.at[pl.ds(mo, _GSUB)],
                                         ignored_value=-1)],
                rows_v.at[pl.ds(dst, _GSUB)], sem)
            cp.start()
            cps.append(cp)
        for cp in cps:
            cp.wait()

        def w_body(w, _):
            wv = jnp.full((16,), 0, jnp.int32) + w
            for i in range(_C // 16):
                v0 = rows_v[w, pl.ds(i * 16, 16)]
                plsc.store_scatter(ob, [wv, ch_idx[0][i]], v0)
                v1 = rows_v[_W + w, pl.ds(i * 16, 16)]
                plsc.store_scatter(ob, [wv, ch_idx[1][i]], v1)
            return 0

        lax.fori_loop(0, _W, w_body, 0)

        pltpu.sync_copy(ob, out_hbm.at[n, h, :, :])
        return 0

    lax.fori_loop(0, _HB, row_body, 0)


_hc_kernel = functools.partial(
    pl.kernel,
    out_type=jax.ShapeDtypeStruct((_N, _H, _W, _CD), jnp.float32),
    mesh=plsc.VectorSubcoreMesh(core_axis_name="c", subcore_axis_name="s"),
    scratch_types=[
        pltpu.VMEM((2 * _RW,), jnp.int32),
        pltpu.VMEM((_CHUNK,), jnp.int32),
        pltpu.VMEM((_D * _W, _C), jnp.float32),
        pltpu.VMEM((_W, _CD), jnp.float32),
        pltpu.SemaphoreType.DMA,
    ],
    compiler_params=pltpu.CompilerParams(
        needs_layout_passes=False, use_tc_tiling_on_sc=True),
)(_hc_body)


def kernel(features, indices):
    out = _hc_kernel(features, indices)
    return jnp.transpose(out, (0, 3, 1, 2))


# speedup vs baseline: 7.2743x; 1.9378x over previous
import functools

import jax
import jax.numpy as jnp
from jax import lax
from jax.experimental import pallas as pl
from jax.experimental.pallas import tpu as pltpu
from jax.experimental.pallas import tpu_sc as plsc

_N, _C, _D, _H, _W = 4, 128, 2, 200, 176
_CD = _C * _D
_HW = _H * _W
_NNZ = 150000
_NC = 2
_NS = 16
_NW = _NC * _NS
_HB = _H // (_NW // _N)
_RW = _HB * _W
_CHUNK = 6000
_NCHUNK = _NNZ // _CHUNK
_QPC = _CHUNK // 16
_AG = 5
_GSUB = 88


def _hc_body(feat_hbm, idx_hbm, out_hbm, map_v, idxb, rows_v, ob, sem):
    wid = lax.axis_index("s") * _NC + lax.axis_index("c")
    n = wid // (_NW // _N)
    hb = wid - n * (_NW // _N)
    base0 = n * (_D * _HW) + hb * _RW
    base1 = base0 + _HW
    iota16 = lax.broadcasted_iota(jnp.int32, (16,), 0)

    def init_body(i, _):
        map_v[pl.ds(i * 16, 16)] = jnp.full((16,), -1, jnp.int32)
        return 0

    lax.fori_loop(0, 2 * _RW // 16, init_body, 0)

    def chunk_body(c, _):
        pltpu.sync_copy(idx_hbm.at[pl.ds(c * _CHUNK, _CHUNK)], idxb)

        def q_body(q, _):
            gs = [idxb[pl.ds((q * _AG + t) * 16, 16)] for t in range(_AG)]
            for t in range(_AG):
                g = gs[t]
                s0 = g - base0
                s1 = g - base1
                in0 = s0.astype(jnp.uint32) < jnp.uint32(_RW)
                in1 = s1.astype(jnp.uint32) < jnp.uint32(_RW)
                m = in0 | in1
                local = jnp.where(in0, s0, s1 + _RW)
                local = jnp.where(m, local, 0)
                vid = (c * _CHUNK + (q * _AG + t) * 16) + iota16
                plsc.store_scatter(map_v, [local], vid, mask=m)
            return 0

        lax.fori_loop(0, _QPC // _AG, q_body, 0)
        return 0

    lax.fori_loop(0, _NCHUNK, chunk_body, 0)

    ch_idx = [[(i * 16 + iota16) * _D + d for i in range(_C // 16)]
              for d in range(_D)]
    zero16 = jnp.zeros((16,), jnp.float32)

    def row_body(r, _):
        h = hb * _HB + r

        @plsc.parallel_loop(0, _W, 1, unroll=2)
        def z_body(w):
            for half in range(2):
                for i in range(_C // 16):
                    rows_v[_W * half + w, pl.ds(i * 16, 16)] = zero16

        cps = []
        for mo, dst in ((r * _W, 0), (r * _W + _GSUB, _GSUB),
                        (_RW + r * _W, _W), (_RW + r * _W + _GSUB, _W + _GSUB)):
            cp = pltpu.make_async_copy(
                feat_hbm.at[plsc.Indices(map_v.at[pl.ds(mo, _GSUB)],
                                         ignored_value=-1)],
                rows_v.at[pl.ds(dst, _GSUB)], sem)
            cp.start()
            cps.append(cp)
        for cp in cps:
            cp.wait()

        @plsc.parallel_loop(0, _W, 1, unroll=2)
        def w_body(w):
            wv = jnp.full((16,), 0, jnp.int32) + w
            for i in range(_C // 16):
                v0 = rows_v[w, pl.ds(i * 16, 16)]
                plsc.store_scatter(ob, [wv, ch_idx[0][i]], v0)
                v1 = rows_v[_W + w, pl.ds(i * 16, 16)]
                plsc.store_scatter(ob, [wv, ch_idx[1][i]], v1)

        pltpu.sync_copy(ob, out_hbm.at[n, h, :, :])
        return 0

    lax.fori_loop(0, _HB, row_body, 0)


_hc_kernel = functools.partial(
    pl.kernel,
    out_type=jax.ShapeDtypeStruct((_N, _H, _W, _CD), jnp.float32),
    mesh=plsc.VectorSubcoreMesh(core_axis_name="c", subcore_axis_name="s"),
    scratch_types=[
        pltpu.VMEM((2 * _RW,), jnp.int32),
        pltpu.VMEM((_CHUNK,), jnp.int32),
        pltpu.VMEM((_D * _W, _C), jnp.float32),
        pltpu.VMEM((_W, _CD), jnp.float32),
        pltpu.SemaphoreType.DMA,
    ],
    compiler_params=pltpu.CompilerParams(
        needs_layout_passes=False, use_tc_tiling_on_sc=True),
)(_hc_body)


def kernel(features, indices):
    out = _hc_kernel(features, indices)
    return jnp.transpose(out, (0, 3, 1, 2))


# speedup vs baseline: 13.0464x; 1.7935x over previous
import functools

import jax
import jax.numpy as jnp
from jax import lax
from jax.experimental import pallas as pl
from jax.experimental.pallas import tpu as pltpu
from jax.experimental.pallas import tpu_sc as plsc

_N, _C, _D, _H, _W = 4, 128, 2, 200, 176
_CD = _C * _D
_HW = _H * _W
_NNZ = 150000
_NC = 2
_NS = 16
_NW = _NC * _NS
_HB = _H // (_NW // _N)
_RW = _HB * _W
_CHUNK = 6000
_NCHUNK = _NNZ // _CHUNK
_QPC = _CHUNK // 16
_AG = 5
_GSUB = 88
_NU = 2 * _HB


def _hc_body(feat_hbm, idx_hbm, out_hbm, map_v, idxa, idxb, rows_a, rows_b,
             ob_a, ob_b, sem_ia, sem_ib, sem_ga, sem_gb, sem_oa, sem_ob):
    wid = lax.axis_index("s") * _NC + lax.axis_index("c")
    n = wid // (_NW // _N)
    hb = wid - n * (_NW // _N)
    base0 = n * (_D * _HW) + hb * _RW
    base1 = base0 + _HW
    iota16 = lax.broadcasted_iota(jnp.int32, (16,), 0)

    def init_body(i, _):
        map_v[pl.ds(i * 16, 16)] = jnp.full((16,), -1, jnp.int32)
        return 0

    lax.fori_loop(0, 2 * _RW // 16, init_body, 0)

    def idx_copy(c, buf, sem):
        return pltpu.make_async_copy(
            idx_hbm.at[pl.ds(c * _CHUNK, _CHUNK)], buf, sem)

    def scan_chunk(c, buf):
        def q_body(q, _):
            gs = [buf[pl.ds((q * _AG + t) * 16, 16)] for t in range(_AG)]
            for t in range(_AG):
                g = gs[t]
                s0 = g - base0
                s1 = g - base1
                in0 = s0.astype(jnp.uint32) < jnp.uint32(_RW)
                in1 = s1.astype(jnp.uint32) < jnp.uint32(_RW)
                m = in0 | in1
                local = jnp.where(in0, s0, s1 + _RW)
                local = jnp.where(m, local, 0)
                vid = (c * _CHUNK + (q * _AG + t) * 16) + iota16
                plsc.store_scatter(map_v, [local], vid, mask=m)
            return 0

        lax.fori_loop(0, _QPC // _AG, q_body, 0)

    idx_copy(0, idxa, sem_ia).start()

    def a_body(t, _):
        a = 2 * t
        b = a + 1

        @pl.when(b < _NCHUNK)
        def _():
            idx_copy(b, idxb, sem_ib).start()

        idx_copy(a, idxa, sem_ia).wait()
        scan_chunk(a, idxa)

        @pl.when(a + 2 < _NCHUNK)
        def _():
            idx_copy(a + 2, idxa, sem_ia).start()

        @pl.when(b < _NCHUNK)
        def _():
            idx_copy(b, idxb, sem_ib).wait()
            scan_chunk(b, idxb)

        return 0

    lax.fori_loop(0, (_NCHUNK + 1) // 2, a_body, 0)

    ch_idx = [[(i * 16 + iota16) * _D + d for i in range(_C // 16)]
              for d in range(_D)]
    zero16 = jnp.zeros((16,), jnp.float32)

    def zero_rows(rows):
        @plsc.parallel_loop(0, _GSUB, 1, unroll=2)
        def z_body(w):
            for half in range(2):
                for i in range(_C // 16):
                    rows[_GSUB * half + w, pl.ds(i * 16, 16)] = zero16

    def gathers(u, rows, sem):
        r = u // 2
        w0 = (u - 2 * r) * _GSUB
        mo = r * _W + w0
        cps = []
        for d in range(2):
            cp = pltpu.make_async_copy(
                feat_hbm.at[plsc.Indices(map_v.at[pl.ds(d * _RW + mo, _GSUB)],
                                         ignored_value=-1)],
                rows.at[pl.ds(d * _GSUB, _GSUB)], sem)
            cps.append(cp)
        return cps

    def out_copy(u, ob, sem):
        r = u // 2
        w0 = (u - 2 * r) * _GSUB
        return pltpu.make_async_copy(
            ob, out_hbm.at[n, hb * _HB + r, pl.ds(w0, _GSUB), :], sem)

    def interleave(rows, ob):
        @plsc.parallel_loop(0, _GSUB, 1, unroll=2)
        def w_body(w):
            wv = jnp.full((16,), 0, jnp.int32) + w
            for i in range(_C // 16):
                v0 = rows[w, pl.ds(i * 16, 16)]
                plsc.store_scatter(ob, [wv, ch_idx[0][i]], v0)
                v1 = rows[_GSUB + w, pl.ds(i * 16, 16)]
                plsc.store_scatter(ob, [wv, ch_idx[1][i]], v1)

    zero_rows(rows_a)
    for cp in gathers(0, rows_a, sem_ga):
        cp.start()

    def b_body(t, _):
        a = 2 * t
        b = a + 1

        zero_rows(rows_b)
        for cp in gathers(b, rows_b, sem_gb):
            cp.start()
        for cp in gathers(a, rows_a, sem_ga):
            cp.wait()

        @pl.when(t > 0)
        def _():
            out_copy(a, ob_a, sem_oa).wait()

        interleave(rows_a, ob_a)
        out_copy(a, ob_a, sem_oa).start()

        @pl.when(a + 2 < _NU)
        def _():
            zero_rows(rows_a)
            for cp in gathers(a + 2, rows_a, sem_ga):
                cp.start()

        for cp in gathers(b, rows_b, sem_gb):
            cp.wait()

        @pl.when(t > 0)
        def _():
            out_copy(b, ob_b, sem_ob).wait()

        interleave(rows_b, ob_b)
        out_copy(b, ob_b, sem_ob).start()
        return 0

    lax.fori_loop(0, _NU // 2, b_body, 0)
    out_copy(_NU - 2, ob_a, sem_oa).wait()
    out_copy(_NU - 1, ob_b, sem_ob).wait()


_hc_kernel = functools.partial(
    pl.kernel,
    out_type=jax.ShapeDtypeStruct((_N, _H, _W, _CD), jnp.float32),
    mesh=plsc.VectorSubcoreMesh(core_axis_name="c", subcore_axis_name="s"),
    scratch_types=[
        pltpu.VMEM((2 * _RW,), jnp.int32),
        pltpu.VMEM((_CHUNK,), jnp.int32),
        pltpu.VMEM((_CHUNK,), jnp.int32),
        pltpu.VMEM((_D * _GSUB, _C), jnp.float32),
        pltpu.VMEM((_D * _GSUB, _C), jnp.float32),
        pltpu.VMEM((_GSUB, _CD), jnp.float32),
        pltpu.VMEM((_GSUB, _CD), jnp.float32),
        pltpu.SemaphoreType.DMA,
        pltpu.SemaphoreType.DMA,
        pltpu.SemaphoreType.DMA,
        pltpu.SemaphoreType.DMA,
        pltpu.SemaphoreType.DMA,
        pltpu.SemaphoreType.DMA,
    ],
    compiler_params=pltpu.CompilerParams(
        needs_layout_passes=False, use_tc_tiling_on_sc=True),
)(_hc_body)


def kernel(features, indices):
    out = _hc_kernel(features, indices)
    return jnp.transpose(out, (0, 3, 1, 2))
